# Initial kernel scaffold; baseline (speedup 1.0000x reference)
import jax
import jax.numpy as jnp
from jax.experimental import pallas as pl


def kernel(x, W1, b1, W2, b2):
    raise NotImplementedError("write your pallas kernel here")


# speedup vs baseline: 15.5923x; 15.5923x over previous
import jax
import jax.numpy as jnp
from jax.experimental import pallas as pl
from jax.experimental.pallas import tpu as pltpu

N_ROWS = 50000
D_IN = 128
HID = 128
C_OUT = 10
F_TRIM = 100
CHUNK = 2000
N_CHUNKS = N_ROWS // CHUNK
BIT_ITERS = 31


def _tm_kernel(x_ref, w1_ref, b1_ref, w2_ref, b2_ref, out_ref, h_ref):
    i = pl.program_id(0)
    xb = x_ref[...]
    h = jnp.maximum(
        jnp.dot(xb, w1_ref[...], preferred_element_type=jnp.float32) + b1_ref[...],
        0.0,
    )
    h_ref[pl.ds(i * CHUNK, CHUNK), :] = h

    @pl.when(i == N_CHUNKS - 1)
    def _tail():
        k1 = jnp.float32(F_TRIM)
        k2 = jnp.float32(N_ROWS - F_TRIM + 1)

        def bis_body(_, st):
            lo1, hi1, lo2, hi2 = st
            mid1 = lo1 + ((hi1 - lo1) >> 1)
            mid2 = lo2 + ((hi2 - lo2) >> 1)
            m1 = jax.lax.bitcast_convert_type(mid1, jnp.float32)
            m2 = jax.lax.bitcast_convert_type(mid2, jnp.float32)

            def cnt_body(c, acc):
                a1, a2 = acc
                hc = h_ref[pl.ds(c * CHUNK, CHUNK), :]
                a1 = a1 + jnp.sum(jnp.where(hc <= m1, 1.0, 0.0), axis=0, keepdims=True)
                a2 = a2 + jnp.sum(jnp.where(hc <= m2, 1.0, 0.0), axis=0, keepdims=True)
                return a1, a2

            z = jnp.zeros((1, HID), jnp.float32)
            c1, c2 = jax.lax.fori_loop(0, N_CHUNKS, cnt_body, (z, z))
            ok1 = c1 >= k1
            ok2 = c2 >= k2
            return (
                jnp.where(ok1, lo1, mid1 + 1),
                jnp.where(ok1, mid1, hi1),
                jnp.where(ok2, lo2, mid2 + 1),
                jnp.where(ok2, mid2, hi2),
            )

        lo0 = jnp.zeros((1, HID), jnp.int32)
        hi0 = jnp.full((1, HID), 0x7F7FFFFF, jnp.int32)
        lo1, _, lo2, _ = jax.lax.fori_loop(
            0, BIT_ITERS, bis_body, (lo0, hi0, lo0, hi0)
        )
        t1 = jax.lax.bitcast_convert_type(lo1, jnp.float32)
        t2 = jax.lax.bitcast_convert_type(lo2, jnp.float32)

        def fin_body(c, acc):
            tot, s1, c1, s2, c2 = acc
            hc = h_ref[pl.ds(c * CHUNK, CHUNK), :]
            tot = tot + jnp.sum(hc, axis=0, keepdims=True)
            lt1 = hc < t1
            s1 = s1 + jnp.sum(jnp.where(lt1, hc, 0.0), axis=0, keepdims=True)
            c1 = c1 + jnp.sum(jnp.where(lt1, 1.0, 0.0), axis=0, keepdims=True)
            lt2 = hc < t2
            s2 = s2 + jnp.sum(jnp.where(lt2, hc, 0.0), axis=0, keepdims=True)
            c2 = c2 + jnp.sum(jnp.where(lt2, 1.0, 0.0), axis=0, keepdims=True)
            return tot, s1, c1, s2, c2

        z = jnp.zeros((1, HID), jnp.float32)
        tot, s1, c1, s2, c2 = jax.lax.fori_loop(
            0, N_CHUNKS, fin_body, (z, z, z, z, z)
        )
        bot = s1 + (k1 - c1) * t1
        top = (tot - s2) - (jnp.float32(N_ROWS) - c2 - k1) * t2
        hbar = (tot - bot - top) * (1.0 / (N_ROWS - 2 * F_TRIM))
        out_ref[...] = (
            jnp.dot(hbar, w2_ref[...], preferred_element_type=jnp.float32)
            + b2_ref[...]
        )


def _run(x, W1, b1, W2p, b2p, interpret=False):
    return pl.pallas_call(
        _tm_kernel,
        grid=(N_CHUNKS,),
        in_specs=[
            pl.BlockSpec((CHUNK, D_IN), lambda i: (i, 0)),
            pl.BlockSpec((D_IN, HID), lambda i: (0, 0)),
            pl.BlockSpec((1, HID), lambda i: (0, 0)),
            pl.BlockSpec((HID, 128), lambda i: (0, 0)),
            pl.BlockSpec((1, 128), lambda i: (0, 0)),
        ],
        out_specs=pl.BlockSpec((1, 128), lambda i: (0, 0)),
        out_shape=jax.ShapeDtypeStruct((1, 128), jnp.float32),
        scratch_shapes=[pltpu.VMEM((N_ROWS, HID), jnp.float32)],
        interpret=interpret,
    )(x, W1, b1, W2p, b2p)


def kernel(x, W1, b1, W2, b2):
    W2p = jnp.zeros((HID, 128), jnp.float32).at[:, :C_OUT].set(W2)
    b2p = jnp.zeros((1, 128), jnp.float32).at[0, :C_OUT].set(b2)
    out = _run(x, W1.astype(jnp.float32), b1.reshape(1, HID), W2p, b2p)
    return out[0, :C_OUT]
